# baseline (speedup 1.0000x reference)
import functools

import jax
import jax.numpy as jnp
from jax.experimental import pallas as pl

_CAP = 80


def _router_kernel(x_ref, w_ref, b_ref,
                   probs_ref, g1_ref, e0_ref, e1_ref, a01_ref,
                   zsum_ref, aux_ref):
    i = pl.program_id(1)
    x = x_ref[0]
    w = w_ref[...]
    b = b_ref[0]
    logits = jnp.dot(x, w, preferred_element_type=jnp.float32) + b[None, :]
    m = jnp.max(logits, axis=-1, keepdims=True)
    ex = jnp.exp(logits - m)
    se = jnp.sum(ex, axis=-1, keepdims=True)
    logp = (logits - m) - jnp.log(se)
    probs = ex / se
    probs_ref[0] = probs

    bt, e = probs.shape
    iota_e = jax.lax.broadcasted_iota(jnp.int32, (bt, e), 1).astype(
        jnp.float32)
    big = jnp.float32(1e9)

    g1 = jnp.max(probs, axis=-1)
    e0 = jnp.min(jnp.where(probs == g1[:, None], iota_e, big), axis=-1)
    sel0 = iota_e == e0[:, None]
    probs2 = jnp.where(sel0, -jnp.float32(1.0), probs)
    g2 = jnp.max(probs2, axis=-1)
    e1 = jnp.min(jnp.where(probs2 == g2[:, None], iota_e, big), axis=-1)
    sel1 = iota_e == e1[:, None]

    g1_ref[0, 0] = g1
    e0_ref[0, 0] = e0
    e1_ref[0, 0] = e1
    a0 = sel0.astype(jnp.float32)
    a1 = sel1.astype(jnp.float32)
    a01_ref[0] = jnp.concatenate([a0, a1], axis=-1)

    zpart = jnp.sum(logp * logp)
    cnt = jnp.sum(a0 + a1, axis=0)
    psum = jnp.sum(probs, axis=0)
    part = jnp.concatenate([cnt[None, :], psum[None, :]], axis=0)

    zpart3 = zpart.reshape(1, 1, 1)

    @pl.when(i == 0)
    def _init():
        zsum_ref[...] = zpart3
        aux_ref[0] = part

    @pl.when(i > 0)
    def _acc():
        zsum_ref[...] += zpart3
        aux_ref[0] += part


def _dispatch_kernel(g1_ref, e0_ref, e1_ref, a01_ref, probs_ref, capb_ref,
                     disp_ref, comb_ref, *, blk, cap):
    i = pl.program_id(1)
    t = g1_ref.shape[2]
    e = probs_ref.shape[2]

    g1row = g1_ref[0, 0]
    gi = g1_ref[0, 0, pl.ds(i * blk, blk)]
    e0t = e0_ref[0, 0, pl.ds(i * blk, blk)]
    e1t = e1_ref[0, 0, pl.ds(i * blk, blk)]
    a01 = a01_ref[0]
    probs = probs_ref[0]

    gt = jnp.broadcast_to(g1row[None, :], (blk, t))
    iota_t = jax.lax.broadcasted_iota(jnp.int32, (blk, t), 1).astype(
        jnp.float32)
    row_id = (i * blk).astype(jnp.float32) + jax.lax.broadcasted_iota(
        jnp.int32, (blk, t), 0).astype(jnp.float32)
    cmp = ((gt > gi[:, None]) |
           ((gt == gi[:, None]) & (iota_t < row_id))).astype(jnp.float32)

    p = jnp.dot(cmp, a01, preferred_element_type=jnp.float32)
    p0 = p[:, :e]
    p1 = p[:, e:]
    count0 = jnp.sum(a01[:, :e], axis=0)

    iota_e = jax.lax.broadcasted_iota(jnp.int32, (blk, e), 1).astype(
        jnp.float32)
    sel0 = iota_e == e0t[:, None]
    sel1 = iota_e == e1t[:, None]
    tp = jnp.where(sel0, p0,
                   jnp.where(sel1, p1 + count0[None, :], -jnp.float32(1.0)))

    valid = (tp >= 0.0) & (tp < capb_ref[0, 0])
    tpm = jnp.where(valid, tp, -jnp.float32(1.0))
    iota_c = jax.lax.broadcasted_iota(jnp.int32, (blk, e, cap), 2).astype(
        jnp.float32)
    d3 = tpm[:, :, None] == iota_c
    d3f = d3.astype(jnp.float32)
    disp_ref[0] = d3
    comb_ref[0] = probs[:, :, None] * d3f


def kernel(token_inputs, W, b, num_experts, expert_capacity):
    x = token_inputs.astype(jnp.float32)
    G, T, D = x.shape
    E = W.shape[1]
    cap = _CAP
    capb = jnp.asarray(expert_capacity, jnp.float32).reshape(1, 1)

    bt = 512
    blk = 256
    nt1 = T // bt
    nt2 = T // blk

    b2 = b.astype(jnp.float32).reshape(1, E)

    probs, g1, e0, e1, a01, zsum, aux = pl.pallas_call(
        _router_kernel,
        grid=(G, nt1),
        in_specs=[
            pl.BlockSpec((1, bt, D), lambda g, i: (g, i, 0)),
            pl.BlockSpec((D, E), lambda g, i: (0, 0)),
            pl.BlockSpec((1, E), lambda g, i: (0, 0)),
        ],
        out_specs=[
            pl.BlockSpec((1, bt, E), lambda g, i: (g, i, 0)),
            pl.BlockSpec((1, 1, bt), lambda g, i: (g, 0, i)),
            pl.BlockSpec((1, 1, bt), lambda g, i: (g, 0, i)),
            pl.BlockSpec((1, 1, bt), lambda g, i: (g, 0, i)),
            pl.BlockSpec((1, bt, 2 * E), lambda g, i: (g, i, 0)),
            pl.BlockSpec((1, 1, 1), lambda g, i: (g, 0, 0)),
            pl.BlockSpec((1, 2, E), lambda g, i: (g, 0, 0)),
        ],
        out_shape=[
            jax.ShapeDtypeStruct((G, T, E), jnp.float32),
            jax.ShapeDtypeStruct((G, 1, T), jnp.float32),
            jax.ShapeDtypeStruct((G, 1, T), jnp.float32),
            jax.ShapeDtypeStruct((G, 1, T), jnp.float32),
            jax.ShapeDtypeStruct((G, T, 2 * E), jnp.float32),
            jax.ShapeDtypeStruct((G, 1, 1), jnp.float32),
            jax.ShapeDtypeStruct((G, 2, E), jnp.float32),
        ],
    )(x, W.astype(jnp.float32), b2)

    dispatch_mask, combine = pl.pallas_call(
        functools.partial(_dispatch_kernel, blk=blk, cap=cap),
        grid=(G, nt2),
        in_specs=[
            pl.BlockSpec((1, 1, T), lambda g, i: (g, 0, 0)),
            pl.BlockSpec((1, 1, T), lambda g, i: (g, 0, 0)),
            pl.BlockSpec((1, 1, T), lambda g, i: (g, 0, 0)),
            pl.BlockSpec((1, T, 2 * E), lambda g, i: (g, 0, 0)),
            pl.BlockSpec((1, blk, E), lambda g, i: (g, i, 0)),
            pl.BlockSpec((1, 1), lambda g, i: (0, 0)),
        ],
        out_specs=[
            pl.BlockSpec((1, blk, E, cap), lambda g, i: (g, i, 0, 0)),
            pl.BlockSpec((1, blk, E, cap), lambda g, i: (g, i, 0, 0)),
        ],
        out_shape=[
            jax.ShapeDtypeStruct((G, T, E, cap), jnp.bool_),
            jax.ShapeDtypeStruct((G, T, E, cap), jnp.float32),
        ],
    )(g1, e0, e1, a01, probs, capb)

    cnt = aux[:, 0, :] / jnp.float32(T)
    psum = aux[:, 1, :] / jnp.float32(T)
    aux_loss = jnp.mean(cnt * psum) * jnp.asarray(num_experts,
                                                  jnp.float32) ** 2
    router_z_loss = jnp.sum(zsum) / jnp.float32(G * T * E)
    return dispatch_mask, combine, aux_loss, router_z_loss


# speedup vs baseline: 1.0088x; 1.0088x over previous
import functools

import jax
import jax.numpy as jnp
from jax.experimental import pallas as pl

_CAP = 80


def _router_kernel(x_ref, w_ref, b_ref,
                   probs_ref, g1_ref, e0_ref, e1_ref, a01_ref,
                   zsum_ref, aux_ref):
    i = pl.program_id(1)
    x = x_ref[0]
    w = w_ref[...]
    b = b_ref[0]
    logits = jnp.dot(x, w, preferred_element_type=jnp.float32) + b[None, :]
    m = jnp.max(logits, axis=-1, keepdims=True)
    ex = jnp.exp(logits - m)
    se = jnp.sum(ex, axis=-1, keepdims=True)
    logp = (logits - m) - jnp.log(se)
    probs = ex / se
    probs_ref[0] = probs

    bt, e = probs.shape
    iota_e = jax.lax.broadcasted_iota(jnp.int32, (bt, e), 1).astype(
        jnp.float32)
    big = jnp.float32(1e9)

    g1 = jnp.max(probs, axis=-1)
    e0 = jnp.min(jnp.where(probs == g1[:, None], iota_e, big), axis=-1)
    sel0 = iota_e == e0[:, None]
    probs2 = jnp.where(sel0, -jnp.float32(1.0), probs)
    g2 = jnp.max(probs2, axis=-1)
    e1 = jnp.min(jnp.where(probs2 == g2[:, None], iota_e, big), axis=-1)
    sel1 = iota_e == e1[:, None]

    g1_ref[0, 0] = g1
    e0_ref[0, 0] = e0
    e1_ref[0, 0] = e1
    a0 = sel0.astype(jnp.float32)
    a1 = sel1.astype(jnp.float32)
    a01_ref[0] = jnp.concatenate([a0, a1], axis=-1)

    zpart = jnp.sum(logp * logp)
    cnt = jnp.sum(a0 + a1, axis=0)
    psum = jnp.sum(probs, axis=0)
    part = jnp.concatenate([cnt[None, :], psum[None, :]], axis=0)

    zpart3 = zpart.reshape(1, 1, 1)

    @pl.when(i == 0)
    def _init():
        zsum_ref[...] = zpart3
        aux_ref[0] = part

    @pl.when(i > 0)
    def _acc():
        zsum_ref[...] += zpart3
        aux_ref[0] += part


def _dispatch_kernel(g1_ref, e0_ref, e1_ref, a01_ref, probs_ref, capb_ref,
                     disp_ref, comb_ref, *, blk, cap):
    i = pl.program_id(1)
    t = g1_ref.shape[2]
    e = probs_ref.shape[2]

    g1row = g1_ref[0, 0]
    gi = g1_ref[0, 0, pl.ds(i * blk, blk)]
    e0t = e0_ref[0, 0, pl.ds(i * blk, blk)]
    e1t = e1_ref[0, 0, pl.ds(i * blk, blk)]
    a01 = a01_ref[0]
    probs = probs_ref[0]

    gt = jnp.broadcast_to(g1row[None, :], (blk, t))
    iota_t = jax.lax.broadcasted_iota(jnp.int32, (blk, t), 1).astype(
        jnp.float32)
    row_id = (i * blk).astype(jnp.float32) + jax.lax.broadcasted_iota(
        jnp.int32, (blk, t), 0).astype(jnp.float32)
    cmp = ((gt > gi[:, None]) |
           ((gt == gi[:, None]) & (iota_t < row_id))).astype(jnp.float32)

    p = jnp.dot(cmp, a01, preferred_element_type=jnp.float32)
    p0 = p[:, :e]
    p1 = p[:, e:]
    count0 = jnp.sum(a01[:, :e], axis=0)

    iota_e = jax.lax.broadcasted_iota(jnp.int32, (blk, e), 1).astype(
        jnp.float32)
    sel0 = iota_e == e0t[:, None]
    sel1 = iota_e == e1t[:, None]
    tp = jnp.where(sel0, p0,
                   jnp.where(sel1, p1 + count0[None, :], -jnp.float32(1.0)))

    valid = (tp >= 0.0) & (tp < capb_ref[0, 0])
    tps = jnp.where(valid, tp + iota_e * jnp.float32(cap), -jnp.float32(1.0))

    ecn = e * cap
    r_i = jax.lax.broadcasted_iota(jnp.int32, (e, ecn), 0)
    c_i = jax.lax.broadcasted_iota(jnp.int32, (e, ecn), 1)
    s = ((c_i >= r_i * cap) & (c_i < r_i * cap + cap)).astype(jnp.float32)

    tpsx = jnp.dot(tps, s, preferred_element_type=jnp.float32,
                   precision=jax.lax.Precision.HIGHEST)
    px = jnp.dot(probs, s, preferred_element_type=jnp.float32,
                 precision=jax.lax.Precision.HIGHEST)
    iec = jax.lax.broadcasted_iota(jnp.int32, (blk, ecn), 1).astype(
        jnp.float32)
    d2 = tpsx == iec
    disp_ref[0] = d2
    comb_ref[0] = px * d2.astype(jnp.float32)


def kernel(token_inputs, W, b, num_experts, expert_capacity):
    x = token_inputs.astype(jnp.float32)
    G, T, D = x.shape
    E = W.shape[1]
    cap = _CAP
    capb = jnp.asarray(expert_capacity, jnp.float32).reshape(1, 1)

    bt = 512
    blk = 256
    nt1 = T // bt
    nt2 = T // blk

    b2 = b.astype(jnp.float32).reshape(1, E)

    probs, g1, e0, e1, a01, zsum, aux = pl.pallas_call(
        _router_kernel,
        grid=(G, nt1),
        in_specs=[
            pl.BlockSpec((1, bt, D), lambda g, i: (g, i, 0)),
            pl.BlockSpec((D, E), lambda g, i: (0, 0)),
            pl.BlockSpec((1, E), lambda g, i: (0, 0)),
        ],
        out_specs=[
            pl.BlockSpec((1, bt, E), lambda g, i: (g, i, 0)),
            pl.BlockSpec((1, 1, bt), lambda g, i: (g, 0, i)),
            pl.BlockSpec((1, 1, bt), lambda g, i: (g, 0, i)),
            pl.BlockSpec((1, 1, bt), lambda g, i: (g, 0, i)),
            pl.BlockSpec((1, bt, 2 * E), lambda g, i: (g, i, 0)),
            pl.BlockSpec((1, 1, 1), lambda g, i: (g, 0, 0)),
            pl.BlockSpec((1, 2, E), lambda g, i: (g, 0, 0)),
        ],
        out_shape=[
            jax.ShapeDtypeStruct((G, T, E), jnp.float32),
            jax.ShapeDtypeStruct((G, 1, T), jnp.float32),
            jax.ShapeDtypeStruct((G, 1, T), jnp.float32),
            jax.ShapeDtypeStruct((G, 1, T), jnp.float32),
            jax.ShapeDtypeStruct((G, T, 2 * E), jnp.float32),
            jax.ShapeDtypeStruct((G, 1, 1), jnp.float32),
            jax.ShapeDtypeStruct((G, 2, E), jnp.float32),
        ],
    )(x, W.astype(jnp.float32), b2)

    dispatch_mask, combine = pl.pallas_call(
        functools.partial(_dispatch_kernel, blk=blk, cap=cap),
        grid=(G, nt2),
        in_specs=[
            pl.BlockSpec((1, 1, T), lambda g, i: (g, 0, 0)),
            pl.BlockSpec((1, 1, T), lambda g, i: (g, 0, 0)),
            pl.BlockSpec((1, 1, T), lambda g, i: (g, 0, 0)),
            pl.BlockSpec((1, T, 2 * E), lambda g, i: (g, 0, 0)),
            pl.BlockSpec((1, blk, E), lambda g, i: (g, i, 0)),
            pl.BlockSpec((1, 1), lambda g, i: (0, 0)),
        ],
        out_specs=[
            pl.BlockSpec((1, blk, E * cap), lambda g, i: (g, i, 0)),
            pl.BlockSpec((1, blk, E * cap), lambda g, i: (g, i, 0)),
        ],
        out_shape=[
            jax.ShapeDtypeStruct((G, T, E * cap), jnp.bool_),
            jax.ShapeDtypeStruct((G, T, E * cap), jnp.float32),
        ],
    )(g1, e0, e1, a01, probs, capb)

    dispatch_mask = dispatch_mask.reshape(G, T, E, cap)
    combine = combine.reshape(G, T, E, cap)
    cnt = aux[:, 0, :] / jnp.float32(T)
    psum = aux[:, 1, :] / jnp.float32(T)
    aux_loss = jnp.mean(cnt * psum) * jnp.asarray(num_experts,
                                                  jnp.float32) ** 2
    router_z_loss = jnp.sum(zsum) / jnp.float32(G * T * E)
    return dispatch_mask, combine, aux_loss, router_z_loss


# speedup vs baseline: 1.3820x; 1.3698x over previous
import functools

import jax
import jax.numpy as jnp
from jax.experimental import pallas as pl

_CAP = 80


def _router_kernel(x_ref, w_ref, b_ref,
                   g1_ref, g2_ref, e0_ref, e1_ref, a01_ref,
                   zsum_ref, aux_ref):
    i = pl.program_id(1)
    x = x_ref[0]
    w = w_ref[...]
    b = b_ref[0]
    logits = jnp.dot(x, w, preferred_element_type=jnp.float32) + b[None, :]
    m = jnp.max(logits, axis=-1, keepdims=True)
    ex = jnp.exp(logits - m)
    se = jnp.sum(ex, axis=-1, keepdims=True)
    logp = (logits - m) - jnp.log(se)
    probs = ex / se

    bt, e = probs.shape
    iota_e = jax.lax.broadcasted_iota(jnp.int32, (bt, e), 1).astype(
        jnp.float32)
    big = jnp.float32(1e9)

    g1 = jnp.max(probs, axis=-1)
    e0 = jnp.min(jnp.where(probs == g1[:, None], iota_e, big), axis=-1)
    sel0 = iota_e == e0[:, None]
    probs2 = jnp.where(sel0, -jnp.float32(1.0), probs)
    g2 = jnp.max(probs2, axis=-1)
    e1 = jnp.min(jnp.where(probs2 == g2[:, None], iota_e, big), axis=-1)
    sel1 = iota_e == e1[:, None]

    g1_ref[0, 0] = g1
    g2_ref[0, 0] = g2
    e0_ref[0, 0] = e0
    e1_ref[0, 0] = e1
    a0 = sel0.astype(jnp.float32)
    a1 = sel1.astype(jnp.float32)
    a01_ref[0] = jnp.concatenate([a0, a1], axis=-1)

    zpart = jnp.sum(logp * logp)
    cnt = jnp.sum(a0 + a1, axis=0)
    psum = jnp.sum(probs, axis=0)
    part = jnp.concatenate([cnt[None, :], psum[None, :]], axis=0)

    zpart3 = zpart.reshape(1, 1, 1)

    @pl.when(i == 0)
    def _init():
        zsum_ref[...] = zpart3
        aux_ref[0] = part

    @pl.when(i > 0)
    def _acc():
        zsum_ref[...] += zpart3
        aux_ref[0] += part


def _dispatch_kernel(g1_ref, g2_ref, e0_ref, e1_ref, a01_ref, capb_ref,
                     disp_ref, comb_ref, *, blk, cap):
    i = pl.program_id(1)
    t = g1_ref.shape[2]
    e = a01_ref.shape[2] // 2

    g1row = g1_ref[0, 0]
    gi = g1_ref[0, 0, pl.ds(i * blk, blk)]
    g2t = g2_ref[0, 0, pl.ds(i * blk, blk)]
    e0t = e0_ref[0, 0, pl.ds(i * blk, blk)]
    e1t = e1_ref[0, 0, pl.ds(i * blk, blk)]
    a01 = a01_ref[0]

    gt = jnp.broadcast_to(g1row[None, :], (blk, t))
    iota_t = jax.lax.broadcasted_iota(jnp.int32, (blk, t), 1).astype(
        jnp.float32)
    row_id = (i * blk).astype(jnp.float32) + jax.lax.broadcasted_iota(
        jnp.int32, (blk, t), 0).astype(jnp.float32)
    cmp = ((gt > gi[:, None]) |
           ((gt == gi[:, None]) & (iota_t < row_id))).astype(jnp.float32)

    p = jnp.dot(cmp, a01, preferred_element_type=jnp.float32)
    p0 = p[:, :e]
    p1 = p[:, e:]
    count0 = jnp.sum(a01[:, :e], axis=0)

    iota_e = jax.lax.broadcasted_iota(jnp.int32, (blk, e), 1).astype(
        jnp.float32)
    sel0 = (iota_e == e0t[:, None]).astype(jnp.float32)
    sel1 = (iota_e == e1t[:, None]).astype(jnp.float32)
    prio0 = jnp.sum(p0 * sel0, axis=-1)
    prio1 = jnp.sum((p1 + count0[None, :]) * sel1, axis=-1)

    capb = capb_ref[0, 0]
    s0 = jnp.where(prio0 < capb, e0t * jnp.float32(cap) + prio0,
                   -jnp.float32(1.0))
    s1 = jnp.where(prio1 < capb, e1t * jnp.float32(cap) + prio1,
                   -jnp.float32(1.0))

    ecn = e * cap
    iec = jax.lax.broadcasted_iota(jnp.int32, (blk, ecn), 1).astype(
        jnp.float32)
    m0 = iec == s0[:, None]
    m1 = iec == s1[:, None]
    disp_ref[0] = m0 | m1
    comb_ref[0] = (gi[:, None] * m0.astype(jnp.float32) +
                   g2t[:, None] * m1.astype(jnp.float32))


def kernel(token_inputs, W, b, num_experts, expert_capacity):
    x = token_inputs.astype(jnp.float32)
    G, T, D = x.shape
    E = W.shape[1]
    cap = _CAP
    capb = jnp.asarray(expert_capacity, jnp.float32).reshape(1, 1)

    bt = 512
    blk = 256
    nt1 = T // bt
    nt2 = T // blk

    b2 = b.astype(jnp.float32).reshape(1, E)

    g1, g2, e0, e1, a01, zsum, aux = pl.pallas_call(
        _router_kernel,
        grid=(G, nt1),
        in_specs=[
            pl.BlockSpec((1, bt, D), lambda g, i: (g, i, 0)),
            pl.BlockSpec((D, E), lambda g, i: (0, 0)),
            pl.BlockSpec((1, E), lambda g, i: (0, 0)),
        ],
        out_specs=[
            pl.BlockSpec((1, 1, bt), lambda g, i: (g, 0, i)),
            pl.BlockSpec((1, 1, bt), lambda g, i: (g, 0, i)),
            pl.BlockSpec((1, 1, bt), lambda g, i: (g, 0, i)),
            pl.BlockSpec((1, 1, bt), lambda g, i: (g, 0, i)),
            pl.BlockSpec((1, bt, 2 * E), lambda g, i: (g, i, 0)),
            pl.BlockSpec((1, 1, 1), lambda g, i: (g, 0, 0)),
            pl.BlockSpec((1, 2, E), lambda g, i: (g, 0, 0)),
        ],
        out_shape=[
            jax.ShapeDtypeStruct((G, 1, T), jnp.float32),
            jax.ShapeDtypeStruct((G, 1, T), jnp.float32),
            jax.ShapeDtypeStruct((G, 1, T), jnp.float32),
            jax.ShapeDtypeStruct((G, 1, T), jnp.float32),
            jax.ShapeDtypeStruct((G, T, 2 * E), jnp.float32),
            jax.ShapeDtypeStruct((G, 1, 1), jnp.float32),
            jax.ShapeDtypeStruct((G, 2, E), jnp.float32),
        ],
    )(x, W.astype(jnp.float32), b2)

    dispatch_mask, combine = pl.pallas_call(
        functools.partial(_dispatch_kernel, blk=blk, cap=cap),
        grid=(G, nt2),
        in_specs=[
            pl.BlockSpec((1, 1, T), lambda g, i: (g, 0, 0)),
            pl.BlockSpec((1, 1, T), lambda g, i: (g, 0, 0)),
            pl.BlockSpec((1, 1, T), lambda g, i: (g, 0, 0)),
            pl.BlockSpec((1, 1, T), lambda g, i: (g, 0, 0)),
            pl.BlockSpec((1, T, 2 * E), lambda g, i: (g, 0, 0)),
            pl.BlockSpec((1, 1), lambda g, i: (0, 0)),
        ],
        out_specs=[
            pl.BlockSpec((1, blk, E * cap), lambda g, i: (g, i, 0)),
            pl.BlockSpec((1, blk, E * cap), lambda g, i: (g, i, 0)),
        ],
        out_shape=[
            jax.ShapeDtypeStruct((G, T, E * cap), jnp.bool_),
            jax.ShapeDtypeStruct((G, T, E * cap), jnp.float32),
        ],
    )(g1, g2, e0, e1, a01, capb)

    dispatch_mask = dispatch_mask.reshape(G, T, E, cap)
    combine = combine.reshape(G, T, E, cap)
    cnt = aux[:, 0, :] / jnp.float32(T)
    psum = aux[:, 1, :] / jnp.float32(T)
    aux_loss = jnp.mean(cnt * psum) * jnp.asarray(num_experts,
                                                  jnp.float32) ** 2
    router_z_loss = jnp.sum(zsum) / jnp.float32(G * T * E)
    return dispatch_mask, combine, aux_loss, router_z_loss
